# Initial kernel scaffold; baseline (speedup 1.0000x reference)
import jax
import jax.numpy as jnp
from jax.experimental import pallas as pl


def kernel(input_x, weight):
    raise NotImplementedError("write your pallas kernel here")


# speedup vs baseline: 1.5011x; 1.5011x over previous
import functools

import jax
import jax.numpy as jnp
from jax import lax
from jax.experimental import pallas as pl
from jax.experimental.pallas import tpu as pltpu
from jax.experimental.pallas import tpu_sc as plsc

B = 4096 * 200
D = 32
NW = 32
B_PER_W = B // NW
CHUNK = 1600
NCHUNK = B_PER_W // CHUNK


def _make_gather_kernel():
    mesh = plsc.VectorSubcoreMesh(core_axis_name="c", subcore_axis_name="s")

    @functools.partial(
        pl.kernel,
        mesh=mesh,
        compiler_params=pltpu.CompilerParams(use_tc_tiling_on_sc=False),
        out_type=jax.ShapeDtypeStruct((B, D), jnp.float32),
        scratch_types=[
            pltpu.VMEM((CHUNK,), jnp.int32),
            pltpu.VMEM((CHUNK,), jnp.int32),
            pltpu.VMEM((CHUNK, D), jnp.float32),
            pltpu.VMEM((CHUNK, D), jnp.float32),
            pltpu.SemaphoreType.DMA,
            pltpu.SemaphoreType.DMA,
        ],
    )
    def gather_kernel(idx_hbm, table_hbm, out_hbm,
                      idx0, idx1, rows0, rows1, gsem, osem):
        wid = lax.axis_index("s") * 2 + lax.axis_index("c")
        base = wid * B_PER_W
        idx_bufs = (idx0, idx1)
        row_bufs = (rows0, rows1)

        def issue(i):
            iv, rv = idx_bufs[i % 2], row_bufs[i % 2]
            off = base + i * CHUNK
            pltpu.sync_copy(idx_hbm.at[pl.ds(off, CHUNK)], iv)
            pltpu.async_copy(table_hbm.at[iv], rv, gsem)

        issue(0)
        for i in range(NCHUNK):
            iv, rv = idx_bufs[i % 2], row_bufs[i % 2]
            if i + 1 < NCHUNK:
                issue(i + 1)
            pltpu.make_async_copy(table_hbm.at[iv], rv, gsem).wait()
            off = base + i * CHUNK
            pltpu.make_async_copy(rv, out_hbm.at[pl.ds(off, CHUNK)],
                                  osem).start()
            if i >= 1:
                pv = row_bufs[(i - 1) % 2]
                pltpu.make_async_copy(
                    pv, out_hbm.at[pl.ds(off - CHUNK, CHUNK)], osem).wait()
        last = base + (NCHUNK - 1) * CHUNK
        pltpu.make_async_copy(row_bufs[(NCHUNK - 1) % 2],
                              out_hbm.at[pl.ds(last, CHUNK)], osem).wait()

    return gather_kernel


_gather = _make_gather_kernel()


def kernel(input_x, weight):
    idx = input_x.reshape(-1).astype(jnp.int32)
    out = _gather(idx, weight)
    return out.reshape(input_x.shape + (weight.shape[1],))


# speedup vs baseline: 1.5016x; 1.0004x over previous
import functools

import jax
import jax.numpy as jnp
from jax import lax
from jax.experimental import pallas as pl
from jax.experimental.pallas import tpu as pltpu
from jax.experimental.pallas import tpu_sc as plsc

B = 4096 * 200
D = 32
NW = 32
B_PER_W = B // NW
CHUNK = 800
NCHUNK = B_PER_W // CHUNK
NBUF = 4


def _make_gather_kernel():
    mesh = plsc.VectorSubcoreMesh(core_axis_name="c", subcore_axis_name="s")

    @functools.partial(
        pl.kernel,
        mesh=mesh,
        compiler_params=pltpu.CompilerParams(use_tc_tiling_on_sc=False),
        out_type=jax.ShapeDtypeStruct((B, D), jnp.float32),
        scratch_types=(
            [pltpu.VMEM((B_PER_W,), jnp.int32)]
            + [pltpu.VMEM((CHUNK, D), jnp.float32) for _ in range(NBUF)]
            + [pltpu.SemaphoreType.DMA, pltpu.SemaphoreType.DMA]
        ),
    )
    def gather_kernel(idx_hbm, table_hbm, out_hbm, idx_all, *rest):
        row_bufs = rest[:NBUF]
        gsem, osem = rest[NBUF], rest[NBUF + 1]
        wid = lax.axis_index("s") * 2 + lax.axis_index("c")
        base = wid * B_PER_W
        pltpu.sync_copy(idx_hbm.at[pl.ds(base, B_PER_W)], idx_all)

        def gather(i):
            rv = row_bufs[i % NBUF]
            pltpu.async_copy(
                table_hbm.at[idx_all.at[pl.ds(i * CHUNK, CHUNK)]], rv, gsem)

        def gwait(i):
            rv = row_bufs[i % NBUF]
            pltpu.make_async_copy(
                table_hbm.at[idx_all.at[pl.ds(i * CHUNK, CHUNK)]], rv,
                gsem).wait()

        def wb_start(i):
            rv = row_bufs[i % NBUF]
            pltpu.make_async_copy(
                rv, out_hbm.at[pl.ds(base + i * CHUNK, CHUNK)], osem).start()

        def wb_wait(i):
            rv = row_bufs[i % NBUF]
            pltpu.make_async_copy(
                rv, out_hbm.at[pl.ds(base + i * CHUNK, CHUNK)], osem).wait()

        for i in range(min(NBUF, NCHUNK)):
            gather(i)
        for i in range(NCHUNK):
            gwait(i)
            wb_start(i)
            if i + NBUF < NCHUNK:
                wb_wait(i)
                gather(i + NBUF)
        for i in range(max(NCHUNK - NBUF, 0), NCHUNK):
            wb_wait(i)

    return gather_kernel


_gather = _make_gather_kernel()


def kernel(input_x, weight):
    idx = input_x.reshape(-1).astype(jnp.int32)
    out = _gather(idx, weight)
    return out.reshape(input_x.shape + (weight.shape[1],))
